# baseline (speedup 1.0000x reference)
import jax
import jax.numpy as jnp
from jax import lax
from jax.experimental import pallas as pl
from jax.experimental.pallas import tpu as pltpu
from jax.experimental.pallas import tpu_sc as plsc

VOCAB = 30528
MAX_POS = 512
TYPE_VOCAB = 2
HIDDEN = 768
B, S = 128, 512
L = 16
NW = 32
POS_PER_W = S // NW
SEGS = HIDDEN // L
EPS = 1e-12


def _rsqrt16(x):
    i = plsc.bitcast(x, jnp.int32)
    i = jnp.full((L,), 0x5F3759DF, dtype=jnp.int32) - lax.shift_right_logical(
        i, jnp.full((L,), 1, dtype=jnp.int32))
    y = plsc.bitcast(i, jnp.float32)
    half = x * 0.5
    for _ in range(3):
        y = y * (1.5 - half * y * y)
    return y


def _body(idsT, ttT, word, pos, typ, gamma, beta, out,
          idsbuf, ttbuf, ptbuf, wrows, obuf, posbuf,
          typebuf, gbuf, bbuf, sem):
    cid = lax.axis_index("c")
    sid = lax.axis_index("s")
    wid = cid * 16 + sid
    base_pos = wid * POS_PER_W

    pltpu.sync_copy(idsT.at[wid], idsbuf)
    pltpu.sync_copy(ttT.at[wid], ttbuf)
    pltpu.sync_copy(pos.at[pl.ds(base_pos, POS_PER_W)], posbuf)
    pltpu.sync_copy(typ, typebuf)
    pltpu.sync_copy(gamma, gbuf)
    pltpu.sync_copy(beta, bbuf)

    for j in range(POS_PER_W):
        def _pt(s, _, j=j):
            sl = pl.ds(s * L, L)
            p = posbuf[j, sl]
            ptbuf[2 * j, sl] = p + typebuf[0, sl]
            ptbuf[2 * j + 1, sl] = p + typebuf[1, sl]
            return 0
        lax.fori_loop(0, SEGS, _pt, 0)

    inv_h = jnp.float32(1.0 / HIDDEN)

    def _batch(b, _):
        ids_vec = idsbuf[pl.ds(b * L, L)]
        pltpu.async_copy(word.at[ids_vec], wrows, sem).wait()
        tt_vec = ttbuf[pl.ds(b * L, L)]

        for j in range(L):
            tt_j = lax.squeeze(lax.slice(tt_vec, (j,), (j + 1,)), (0,))
            row = 2 * j + tt_j

            def _p1(s, carry, j=j, row=row):
                acc, sq = carry
                sl = pl.ds(s * L, L)
                y = wrows[j, sl] + ptbuf[row, sl]
                obuf[j, sl] = y
                return acc + y, sq + y * y
            zero = jnp.zeros((L,), jnp.float32)
            acc, sq = lax.fori_loop(0, SEGS, _p1, (zero, zero))
            mu = jnp.sum(acc) * inv_h
            var = jnp.sum(sq) * inv_h - mu * mu
            var = jnp.maximum(var, 0.0) + EPS
            rstd = _rsqrt16(jnp.full((L,), var))
            nmr = jnp.full((L,), -mu) * rstd

            def _p2(s, _, j=j, rstd=rstd, nmr=nmr):
                sl = pl.ds(s * L, L)
                y = obuf[j, sl] * rstd + nmr
                obuf[j, sl] = y * gbuf[sl] + bbuf[sl]
                return 0
            lax.fori_loop(0, SEGS, _p2, 0)

        pltpu.sync_copy(obuf, out.at[pl.ds(b * S + base_pos, L)])
        return 0

    lax.fori_loop(0, B, _batch, 0)


@jax.jit
def kernel(input_ids, token_type_ids, word_emb, pos_emb, type_emb, gamma, beta):
    ids = input_ids.astype(jnp.int32)
    tt = token_type_ids.astype(jnp.int32)
    idsT = ids.reshape(B, NW, POS_PER_W).transpose(1, 0, 2).reshape(NW, B * POS_PER_W)
    ttT = tt.reshape(B, NW, POS_PER_W).transpose(1, 0, 2).reshape(NW, B * POS_PER_W)

    run = pl.kernel(
        _body,
        out_type=jax.ShapeDtypeStruct((B * S, HIDDEN), jnp.float32),
        mesh=plsc.VectorSubcoreMesh(core_axis_name="c", subcore_axis_name="s"),
        scratch_types=[
            pltpu.VMEM((B * POS_PER_W,), jnp.int32),
            pltpu.VMEM((B * POS_PER_W,), jnp.int32),
            pltpu.VMEM((2 * POS_PER_W, HIDDEN), jnp.float32),
            pltpu.VMEM((L, HIDDEN), jnp.float32),
            pltpu.VMEM((L, HIDDEN), jnp.float32),
            pltpu.VMEM((POS_PER_W, HIDDEN), jnp.float32),
            pltpu.VMEM((TYPE_VOCAB, HIDDEN), jnp.float32),
            pltpu.VMEM((HIDDEN,), jnp.float32),
            pltpu.VMEM((HIDDEN,), jnp.float32),
            pltpu.SemaphoreType.DMA,
        ],
        compiler_params=pltpu.CompilerParams(needs_layout_passes=False),
    )
    out = run(idsT, ttT, word_emb, pos_emb, type_emb, gamma, beta)
    return out.reshape(B, S, HIDDEN)


# speedup vs baseline: 1.1458x; 1.1458x over previous
import jax
import jax.numpy as jnp
from jax import lax
from jax.experimental import pallas as pl
from jax.experimental.pallas import tpu as pltpu
from jax.experimental.pallas import tpu_sc as plsc

VOCAB = 30528
MAX_POS = 512
TYPE_VOCAB = 2
HIDDEN = 768
B, S = 128, 512
L = 16
NW = 32
POS_PER_W = S // NW
SEGS = HIDDEN // L
EPS = 1e-12


def _rsqrt16(x):
    i = plsc.bitcast(x, jnp.int32)
    i = jnp.full((L,), 0x5F3759DF, dtype=jnp.int32) - lax.shift_right_logical(
        i, jnp.full((L,), 1, dtype=jnp.int32))
    y = plsc.bitcast(i, jnp.float32)
    half = x * 0.5
    for _ in range(3):
        y = y * (1.5 - half * y * y)
    return y


def _body(idsT, ttT, word, pos, typ, gamma, beta, out,
          idsbuf, ttbuf, ptbuf, wrows0, wrows1, obuf0, obuf1, posbuf,
          typebuf, gbuf, bbuf, sem_in0, sem_in1, sem_out0, sem_out1):
    cid = lax.axis_index("c")
    sid = lax.axis_index("s")
    wid = cid * 16 + sid
    base_pos = wid * POS_PER_W

    pltpu.sync_copy(idsT.at[wid], idsbuf)
    pltpu.sync_copy(ttT.at[wid], ttbuf)
    pltpu.sync_copy(pos.at[pl.ds(base_pos, POS_PER_W)], posbuf)
    pltpu.sync_copy(typ, typebuf)
    pltpu.sync_copy(gamma, gbuf)
    pltpu.sync_copy(beta, bbuf)

    for j in range(POS_PER_W):
        def _pt(s, _, j=j):
            sl = pl.ds(s * L, L)
            p = posbuf[j, sl]
            ptbuf[2 * j, sl] = p + typebuf[0, sl]
            ptbuf[2 * j + 1, sl] = p + typebuf[1, sl]
            return 0
        lax.fori_loop(0, SEGS, _pt, 0)

    inv_h = jnp.float32(1.0 / HIDDEN)

    def _gather_start(b, wr, s_in):
        ids_vec = idsbuf[pl.ds(b * L, L)]
        pltpu.async_copy(word.at[ids_vec], wr, s_in)

    def _gather_wait(wr, s_in):
        dummy = jnp.zeros((L,), jnp.int32)
        pltpu.make_async_copy(word.at[dummy], wr, s_in).wait()

    def _out_wait(ob, s_out):
        pltpu.make_async_copy(ob, out.at[pl.ds(base_pos, L)], s_out).wait()

    def _compute(b, wr, ob):
        tt_vec = ttbuf[pl.ds(b * L, L)]
        for j in range(L):
            tt_j = lax.squeeze(lax.slice(tt_vec, (j,), (j + 1,)), (0,))
            row = 2 * j + tt_j

            def _p1(s, carry, j=j, row=row):
                acc, sq = carry
                sl = pl.ds(s * L, L)
                y = wr[j, sl] + ptbuf[row, sl]
                ob[j, sl] = y
                return acc + y, sq + y * y
            zero = jnp.zeros((L,), jnp.float32)
            acc, sq = lax.fori_loop(0, SEGS, _p1, (zero, zero))
            mu = jnp.sum(acc) * inv_h
            var = jnp.sum(sq) * inv_h - mu * mu
            var = jnp.maximum(var, 0.0) + EPS
            rstd = _rsqrt16(jnp.full((L,), var))
            nmr = jnp.full((L,), -mu) * rstd

            def _p2(s, _, j=j, rstd=rstd, nmr=nmr):
                sl = pl.ds(s * L, L)
                y = ob[j, sl] * rstd + nmr
                ob[j, sl] = y * gbuf[sl] + bbuf[sl]
                return 0
            lax.fori_loop(0, SEGS, _p2, 0)

    wrs = (wrows0, wrows1)
    obs = (obuf0, obuf1)
    sin = (sem_in0, sem_in1)
    sout = (sem_out0, sem_out1)

    _gather_start(0, wrs[0], sin[0])

    def _pair(b, _):
        for ph in range(2):
            bb = b + ph

            @pl.when(bb + 1 < B)
            def _():
                _gather_start(bb + 1, wrs[1 - ph], sin[1 - ph])

            _gather_wait(wrs[ph], sin[ph])

            @pl.when(bb >= 2)
            def _():
                _out_wait(obs[ph], sout[ph])

            _compute(bb, wrs[ph], obs[ph])
            pltpu.async_copy(obs[ph], out.at[pl.ds(bb * S + base_pos, L)],
                             sout[ph])
        return 0

    lax.fori_loop(0, B // 2, lambda i, c: _pair(i * 2, c), 0)
    _out_wait(obs[0], sout[0])
    _out_wait(obs[1], sout[1])


@jax.jit
def kernel(input_ids, token_type_ids, word_emb, pos_emb, type_emb, gamma, beta):
    ids = input_ids.astype(jnp.int32)
    tt = token_type_ids.astype(jnp.int32)
    idsT = ids.reshape(B, NW, POS_PER_W).transpose(1, 0, 2).reshape(NW, B * POS_PER_W)
    ttT = tt.reshape(B, NW, POS_PER_W).transpose(1, 0, 2).reshape(NW, B * POS_PER_W)

    run = pl.kernel(
        _body,
        out_type=jax.ShapeDtypeStruct((B * S, HIDDEN), jnp.float32),
        mesh=plsc.VectorSubcoreMesh(core_axis_name="c", subcore_axis_name="s"),
        scratch_types=[
            pltpu.VMEM((B * POS_PER_W,), jnp.int32),
            pltpu.VMEM((B * POS_PER_W,), jnp.int32),
            pltpu.VMEM((2 * POS_PER_W, HIDDEN), jnp.float32),
            pltpu.VMEM((L, HIDDEN), jnp.float32),
            pltpu.VMEM((L, HIDDEN), jnp.float32),
            pltpu.VMEM((L, HIDDEN), jnp.float32),
            pltpu.VMEM((L, HIDDEN), jnp.float32),
            pltpu.VMEM((POS_PER_W, HIDDEN), jnp.float32),
            pltpu.VMEM((TYPE_VOCAB, HIDDEN), jnp.float32),
            pltpu.VMEM((HIDDEN,), jnp.float32),
            pltpu.VMEM((HIDDEN,), jnp.float32),
            pltpu.SemaphoreType.DMA,
            pltpu.SemaphoreType.DMA,
            pltpu.SemaphoreType.DMA,
            pltpu.SemaphoreType.DMA,
        ],
        compiler_params=pltpu.CompilerParams(needs_layout_passes=False),
    )
    out = run(idsT, ttT, word_emb, pos_emb, type_emb, gamma, beta)
    return out.reshape(B, S, HIDDEN)
